# baseline (speedup 1.0000x reference)
import jax
import jax.numpy as jnp
from jax import lax
from jax.experimental import pallas as pl
from jax.experimental.pallas import tpu as pltpu
from jax.experimental.pallas import tpu_sc as plsc

_H = 16
_V = 32
_N = 1024
_TOTAL = _N * _N
_NC = 2
_NS = 16
_LANES = 16
_NW = _NC * _NS
_ROWS_W = _N // _NW
_R = 2
_NSTEP = _ROWS_W // _R
_CHUNK = _R * _N
_STRIDE = _V + 1
_LUT_ROW = 640


def _gdb_body(d_hbm, tabT_hbm, out_hbm, cols_v, d_v, o_v,
              dsem0, dsem1, osem0, osem1):
    wid = lax.axis_index("s") * _NC + lax.axis_index("c")
    row_w = wid * _ROWS_W
    dsems = (dsem0, dsem1)
    osems = (osem0, osem1)

    pltpu.sync_copy(tabT_hbm, cols_v)

    def start_d(g, b):
        r0 = row_w + g * _R
        return pltpu.async_copy(
            d_hbm.at[pl.ds(r0, _R), :], d_v.at[b], dsems[b])

    def start_o_half(g, b, hp):
        r0 = row_w + g * _R
        return pltpu.async_copy(
            o_v.at[b, pl.ds(8 * hp, 8)],
            out_hbm.at[pl.ds(8 * hp, 8), pl.ds(r0, _R), :], osems[b])

    def wait_d(b):
        pltpu.make_async_copy(
            d_hbm.at[pl.ds(0, _R), :], d_v.at[b], dsems[b]).wait()

    def wait_o(b):
        for hp in range(2):
            pltpu.make_async_copy(
                o_v.at[b, pl.ds(8 * hp, 8)],
                out_hbm.at[pl.ds(8 * hp, 8), pl.ds(0, _R), :],
                osems[b]).wait()

    plo = [cols_v[h, pl.ds(0, _LANES)] for h in range(_H)]
    phi = [cols_v[h, pl.ds(_LANES, _LANES)] for h in range(_H)]
    gdn = lax.GatherDimensionNumbers(
        offset_dims=(), collapsed_slice_dims=(0,), start_index_map=(0,))

    def dg16(tab, idx):
        return lax.gather(
            tab, idx[:, None], dimension_numbers=gdn, slice_sizes=(1,),
            mode=lax.GatherScatterMode.PROMISE_IN_BOUNDS)

    def compute(g, b):
        for hp in range(2):
            for r in range(_R):
                def slice_body(s, c, hp=hp, r=r):
                    off = s * _LANES
                    idx = d_v[b, r, pl.ds(off, _LANES)]
                    idx15 = jnp.bitwise_and(idx, _LANES - 1)
                    m = idx < _LANES
                    for h in range(8 * hp, 8 * hp + 8):
                        v = jnp.where(
                            m, dg16(plo[h], idx15), dg16(phi[h], idx15))
                        o_v[b, h, r, pl.ds(off, _LANES)] = v
                    return c
                lax.fori_loop(0, _N // _LANES, slice_body, 0, unroll=16)
            start_o_half(g, b, hp)

    start_d(0, 0)
    start_d(1, 1)

    def pair_body(g0, c):
        for b in range(2):
            g = 2 * g0 + b
            wait_d(b)

            @pl.when(g >= 2)
            def _():
                wait_o(b)

            compute(g, b)

            @pl.when(g + 2 < _NSTEP)
            def _():
                start_d(g + 2, b)
        return c

    lax.fori_loop(0, _NSTEP // 2, pair_body, 0)
    wait_o(0)
    wait_o(1)


def kernel(distances, table):
    d_2d = distances.astype(jnp.int32)
    tab_t = table.T.reshape(_H, _V)

    mesh = plsc.VectorSubcoreMesh(
        core_axis_name="c", subcore_axis_name="s",
        num_cores=_NC, num_subcores=_NS)

    run = pl.kernel(
        _gdb_body,
        out_type=jax.ShapeDtypeStruct((_H, _N, _N), jnp.float32),
        mesh=mesh,
        scratch_types=[
            pltpu.VMEM((_H, _V), jnp.float32),
            pltpu.VMEM((2, _R, _N), jnp.int32),
            pltpu.VMEM((2, _H, _R, _N), jnp.float32),
            pltpu.SemaphoreType.DMA,
            pltpu.SemaphoreType.DMA,
            pltpu.SemaphoreType.DMA,
            pltpu.SemaphoreType.DMA,
        ],
        compiler_params=pltpu.CompilerParams(needs_layout_passes=False),
    )
    return run(d_2d, tab_t)


# speedup vs baseline: 1.9571x; 1.9571x over previous
import jax
import jax.numpy as jnp
from jax import lax
from jax.experimental import pallas as pl
from jax.experimental.pallas import tpu as pltpu
from jax.experimental.pallas import tpu_sc as plsc

_H = 16
_V = 32
_N = 1024
_NC = 2
_NS = 16
_LANES = 16
_NW = _NC * _NS
_ROWS_W = _N // _NW
_R = 2
_NSTEP = _ROWS_W // _R


def _gdb_body(d_hbm, tabT_hbm, out_hbm, cols_v, d_v, o_v,
              dsem0, dsem1, osem0, osem1):
    wid = lax.axis_index("s") * _NC + lax.axis_index("c")
    row_w = wid * _ROWS_W
    dsems = (dsem0, dsem1)
    osems = (osem0, osem1)

    pltpu.sync_copy(tabT_hbm, cols_v)

    def start_d(g, b):
        r0 = row_w + g * _R
        return pltpu.async_copy(
            d_hbm.at[pl.ds(r0, _R), :], d_v.at[b], dsems[b])

    def start_o_half(g, b, hp):
        r0 = row_w + g * _R
        return pltpu.async_copy(
            o_v.at[b, pl.ds(8 * hp, 8)],
            out_hbm.at[pl.ds(8 * hp, 8), pl.ds(r0, _R), :], osems[b])

    def wait_d(b):
        pltpu.make_async_copy(
            d_hbm.at[pl.ds(0, _R), :], d_v.at[b], dsems[b]).wait()

    def wait_o(b):
        for hp in range(2):
            pltpu.make_async_copy(
                o_v.at[b, pl.ds(8 * hp, 8)],
                out_hbm.at[pl.ds(8 * hp, 8), pl.ds(0, _R), :],
                osems[b]).wait()

    plo = [cols_v[h, pl.ds(0, _LANES)] for h in range(_H)]
    phi = [cols_v[h, pl.ds(_LANES, _LANES)] for h in range(_H)]
    gdn = lax.GatherDimensionNumbers(
        offset_dims=(), collapsed_slice_dims=(0,), start_index_map=(0,))

    def dg16(tab, idx):
        return lax.gather(
            tab, idx[:, None], dimension_numbers=gdn, slice_sizes=(1,),
            mode=lax.GatherScatterMode.PROMISE_IN_BOUNDS)

    def compute(g, b):
        for hp in range(2):
            for r in range(_R):
                def slice_body(s, c, hp=hp, r=r):
                    off = s * _LANES
                    idx = d_v[b, r, pl.ds(off, _LANES)]
                    idx15 = jnp.bitwise_and(idx, _LANES - 1)
                    m = idx < _LANES
                    for h in range(8 * hp, 8 * hp + 8):
                        v = jnp.where(
                            m, dg16(plo[h], idx15), dg16(phi[h], idx15))
                        o_v[b, h, r, pl.ds(off, _LANES)] = v
                    return c
                lax.fori_loop(0, _N // _LANES, slice_body, 0, unroll=8)
            start_o_half(g, b, hp)

    start_d(0, 0)
    start_d(1, 1)

    def pair_body(g0, c):
        for b in range(2):
            g = 2 * g0 + b
            wait_d(b)

            @pl.when(g >= 2)
            def _():
                wait_o(b)

            compute(g, b)

            @pl.when(g + 2 < _NSTEP)
            def _():
                start_d(g + 2, b)
        return c

    lax.fori_loop(0, _NSTEP // 2, pair_body, 0)
    wait_o(0)
    wait_o(1)


def kernel(distances, table):
    d_2d = distances.astype(jnp.int32)
    tab_t = table.T.reshape(_H, _V)

    mesh = plsc.VectorSubcoreMesh(
        core_axis_name="c", subcore_axis_name="s",
        num_cores=_NC, num_subcores=_NS)

    run = pl.kernel(
        _gdb_body,
        out_type=jax.ShapeDtypeStruct((_H, _N, _N), jnp.float32),
        mesh=mesh,
        scratch_types=[
            pltpu.VMEM((_H, _V), jnp.float32),
            pltpu.VMEM((2, _R, _N), jnp.int32),
            pltpu.VMEM((2, _H, _R, _N), jnp.float32),
            pltpu.SemaphoreType.DMA,
            pltpu.SemaphoreType.DMA,
            pltpu.SemaphoreType.DMA,
            pltpu.SemaphoreType.DMA,
        ],
        compiler_params=pltpu.CompilerParams(needs_layout_passes=False),
    )
    return run(d_2d, tab_t)
